# Initial kernel scaffold; baseline (speedup 1.0000x reference)
import jax
import jax.numpy as jnp
from jax.experimental import pallas as pl


def kernel(x, edge_index, edge_attr, Wdown, bdown, Wtime, btime, Wq, bq, Wk, bk, Wv, bv, cluster_emb, Wout, bout, Wup, bup):
    raise NotImplementedError("write your pallas kernel here")


# speedup vs baseline: 16.8579x; 16.8579x over previous
import functools

import jax
import jax.numpy as jnp
from jax import lax
from jax.experimental import pallas as pl
from jax.experimental.pallas import tpu as pltpu
from jax.experimental.pallas import tpu_sc as plsc

N = 10000
E = 320000
EP = 327680
EPR = EP // 128
D = 64
NCL = 8
S1R = 80128
TR = 10016
NW = 32
EW = EP // NW
SCAL = D ** -0.5

f32 = jnp.float32
i32 = jnp.int32


def _mesh():
    return plsc.VectorSubcoreMesh(
        core_axis_name="c", subcore_axis_name="s", num_cores=2, num_subcores=16)


def _node_dense_body(x_ref, Wd_ref, bd_ref, Wq_ref, bq_ref, WkT_ref, bkc_ref,
                     qwb_ref):
    nf = jnp.maximum(
        jnp.dot(x_ref[...], Wd_ref[...], precision="highest") + bd_ref[...], 0.0)
    q = jnp.dot(nf, Wq_ref[...], precision="highest") + bq_ref[...]
    qW = jnp.dot(q, WkT_ref[...], precision="highest") * SCAL
    qb = jnp.dot(q, bkc_ref[...], precision="highest") * SCAL
    qwb_ref[...] = jnp.concatenate(
        [qW, qb, jnp.zeros((N, 15), f32)], axis=1)


def _node_dense(x, Wd, bd, Wq, bq, WkT, bkc):
    return pl.pallas_call(
        _node_dense_body,
        out_shape=jax.ShapeDtypeStruct((N, 80), f32),
    )(x, Wd, bd, Wq, bq, WkT, bkc)


def _sc_gather(qwb_p, src2):
    @functools.partial(
        pl.kernel,
        out_type=jax.ShapeDtypeStruct((EP, 80), f32),
        mesh=_mesh(),
        compiler_params=pltpu.CompilerParams(use_tc_tiling_on_sc=False, needs_layout_passes=False),
        scratch_types=[
            pltpu.VMEM((8, 128), i32),
            pltpu.VMEM((1024, 80), f32),
            pltpu.SemaphoreType.DMA,
        ],
    )
    def k(qwb_h, src_h, out_h, sidx, rows, sem):
        wid = lax.axis_index("c") * 16 + lax.axis_index("s")

        def chunk(j, t):
            row0 = wid * 80 + j * 8
            pltpu.sync_copy(src_h.at[pl.ds(row0, 8)], sidx)
            descs = [
                pltpu.async_copy(qwb_h.at[sidx.at[b]],
                                 rows.at[pl.ds(b * 128, 128)], sem)
                for b in range(8)
            ]
            for dsc in descs:
                dsc.wait()
            pltpu.sync_copy(rows, out_h.at[pl.ds(row0 * 128, 1024)])
            return t

        lax.fori_loop(0, 10, chunk, 0)

    return k(qwb_p, src2)


EB = 8192
GB = EP // EB


def _edge_dense_body(ea_ref, qs_ref, Wt_ref, bt_ref, ce_ref,
                     tf_ref, asg_ref, attn_ref, m_ref):
    i = pl.program_id(0)
    tf = jnp.maximum(
        jnp.dot(ea_ref[...], Wt_ref[...], precision="highest") + bt_ref[...], 0.0)
    tf_ref[...] = tf
    sims = jnp.dot(tf, ce_ref[...], precision="highest")
    best = sims[:, 0:1]
    bi = jnp.zeros((EB, 1), i32)
    for c in range(1, NCL):
        sc = sims[:, c:c + 1]
        m = sc > best
        bi = jnp.where(m, c, bi)
        best = jnp.where(m, sc, best)
    qs = qs_ref[...]
    attn = jnp.sum(qs[:, :D] * tf, axis=1, keepdims=True) + qs[:, D:D + 1]
    asg_ref[...] = bi.reshape(EB // 128, 128)
    attn_ref[...] = attn.reshape(EB // 128, 128)

    @pl.when(i == 0)
    def _():
        m_ref[0, 0] = -1e30

    m_ref[0, 0] = jnp.maximum(m_ref[0, 0], jnp.max(attn))


def _edge_dense(ea_p, qsrc, Wt, bt, ceT):
    rb = EB // 128
    return pl.pallas_call(
        _edge_dense_body,
        grid=(GB,),
        in_specs=[
            pl.BlockSpec((EB, 16), lambda i: (i, 0)),
            pl.BlockSpec((EB, 80), lambda i: (i, 0)),
            pl.BlockSpec((16, D), lambda i: (0, 0)),
            pl.BlockSpec((1, D), lambda i: (0, 0)),
            pl.BlockSpec((D, NCL), lambda i: (0, 0)),
        ],
        out_specs=[
            pl.BlockSpec((EB, D), lambda i: (i, 0)),
            pl.BlockSpec((rb, 128), lambda i: (i, 0)),
            pl.BlockSpec((rb, 128), lambda i: (i, 0)),
            pl.BlockSpec(memory_space=pltpu.MemorySpace.SMEM),
        ],
        out_shape=[
            jax.ShapeDtypeStruct((EP, D), f32),
            jax.ShapeDtypeStruct((EPR, 128), i32),
            jax.ShapeDtypeStruct((EPR, 128), f32),
            jax.ShapeDtypeStruct((1, 1), f32),
        ],
    )(ea_p, qsrc, Wt, bt, ceT)


def _sc_scatter1(attn2, src2, asg2, mv):
    @functools.partial(
        pl.kernel,
        out_type=(
            jax.ShapeDtypeStruct((2, S1R, 16), f32),
            jax.ShapeDtypeStruct((EPR, 128), f32),
        ),
        mesh=_mesh(),
        compiler_params=pltpu.CompilerParams(use_tc_tiling_on_sc=False, needs_layout_passes=False),
        scratch_types=[
            pltpu.VMEM_SHARED((S1R, 16), f32),
            pltpu.VMEM((8, 128), f32),
            pltpu.VMEM((8, 128), i32),
            pltpu.VMEM((8, 128), i32),
            pltpu.VMEM((8, 128), f32),
            pltpu.VMEM((8, 128), i32),
            pltpu.VMEM((1024, 16), f32),
            pltpu.VMEM((1, 128), f32),
        ],
    )
    def k(attn_h, src_h, asg_h, mv_h, s1_h, ex_h,
          s1s, abuf, sbuf, cbuf, exbuf, idxb, rowb, mb):
        cc = lax.axis_index("c")
        ss = lax.axis_index("s")
        wid = cc * 16 + ss
        zero16 = jnp.zeros((16,), f32)
        tmpl = jnp.where(lax.iota(i32, 16) == 1, 1.0, 0.0).astype(f32)

        def zb_init(i, t):
            rowb[i, :] = zero16
            return t

        lax.fori_loop(0, 1024, zb_init, 0)
        for p in range(4):
            pltpu.sync_copy(rowb, s1s.at[pl.ds(ss * 5008 + p * 1024, 1024)])
        pltpu.sync_copy(rowb.at[pl.ds(0, 912)],
                        s1s.at[pl.ds(ss * 5008 + 4096, 912)])

        def rb_init(i, t):
            rowb[i, :] = tmpl
            return t

        lax.fori_loop(0, 1024, rb_init, 0)
        pltpu.sync_copy(mv_h, mb)
        plsc.subcore_barrier()
        mvec = mb[0, pl.ds(0, 16)]
        lanes = lax.iota(i32, 16)
        zlanes = jnp.zeros((16,), i32)

        def chunk(j, t):
            row0 = wid * 80 + j * 8
            pltpu.sync_copy(attn_h.at[pl.ds(row0, 8)], abuf)
            pltpu.sync_copy(src_h.at[pl.ds(row0, 8)], sbuf)
            pltpu.sync_copy(asg_h.at[pl.ds(row0, 8)], cbuf)
            for g in range(64):
                b, off = g // 8, (g % 8) * 16
                ex = jnp.exp(abuf[b, pl.ds(off, 16)] - mvec)
                exbuf[b, pl.ds(off, 16)] = ex
                idxb[b, pl.ds(off, 16)] = (
                    sbuf[b, pl.ds(off, 16)] * 8 + cbuf[b, pl.ds(off, 16)])
                plsc.store_scatter(rowb, [g * 16 + lanes, zlanes], ex)
            pltpu.sync_copy(exbuf, ex_h.at[pl.ds(row0, 8)])
            for b in range(8):
                pltpu.sync_copy(rowb.at[pl.ds(b * 128, 128)],
                                s1s.at[idxb.at[b]], add=True)
            return t

        lax.fori_loop(0, 10, chunk, 0)
        plsc.subcore_barrier()
        pltpu.sync_copy(s1s.at[pl.ds(ss * 5008, 5008)],
                        s1_h.at[cc, pl.ds(ss * 5008, 5008)])

    return k(attn2, src2, asg2, mv)


def _scale_body(es_ref, cn_ref, scaleT_ref, bscal_ref):
    es = es_ref[0] + es_ref[1]
    cn = cn_ref[0] + cn_ref[1]
    g = (jnp.sum(cn, axis=1, keepdims=True) > 0).astype(f32)
    nne = jnp.sum(g)
    mc = jnp.maximum(cn, 1.0)
    scaleT_ref[...] = g / (jnp.where(es > 0, es, 1.0) * mc * nne)
    bscal_ref[...] = jnp.sum(
        g * (cn > 0).astype(f32) / (mc * nne), axis=0, keepdims=True)


def _scale(es2, cn2):
    return pl.pallas_call(
        _scale_body,
        out_shape=[
            jax.ShapeDtypeStruct((NCL, N), f32),
            jax.ShapeDtypeStruct((1, N), f32),
        ],
    )(es2, cn2)


def _sc_wgather(scalef, ex2, src2, asg2):
    @functools.partial(
        pl.kernel,
        out_type=jax.ShapeDtypeStruct((EPR, 128), f32),
        mesh=_mesh(),
        compiler_params=pltpu.CompilerParams(use_tc_tiling_on_sc=False, needs_layout_passes=False),
        scratch_types=[
            pltpu.VMEM((S1R,), f32),
            pltpu.VMEM((16, 128), f32),
            pltpu.VMEM((16, 128), i32),
            pltpu.VMEM((16, 128), i32),
            pltpu.VMEM((16, 128), f32),
        ],
    )
    def k(scale_h, ex_h, src_h, asg_h, w_h, scv, exb, sb, cb, wb):
        wid = lax.axis_index("c") * 16 + lax.axis_index("s")
        pltpu.sync_copy(scale_h, scv)

        def chunk(j, t):
            row0 = wid * 80 + j * 16
            pltpu.sync_copy(ex_h.at[pl.ds(row0, 16)], exb)
            pltpu.sync_copy(src_h.at[pl.ds(row0, 16)], sb)
            pltpu.sync_copy(asg_h.at[pl.ds(row0, 16)], cb)
            for g in range(128):
                b, off = g // 8, (g % 8) * 16
                idx = sb[b, pl.ds(off, 16)] * 8 + cb[b, pl.ds(off, 16)]
                sc16 = plsc.load_gather(scv, [idx])
                wb[b, pl.ds(off, 16)] = exb[b, pl.ds(off, 16)] * sc16
            pltpu.sync_copy(wb, w_h.at[pl.ds(row0, 16)])
            return t

        lax.fori_loop(0, 5, chunk, 0)

    return k(scalef, ex2, src2, asg2)


def _sc_scatter2(w2, src2, tf):
    @functools.partial(
        pl.kernel,
        out_type=jax.ShapeDtypeStruct((2, TR, D), f32),
        mesh=_mesh(),
        compiler_params=pltpu.CompilerParams(use_tc_tiling_on_sc=False, needs_layout_passes=False),
        scratch_types=[
            pltpu.VMEM_SHARED((TR, D), f32),
            pltpu.VMEM((512, D), f32),
            pltpu.VMEM((4, 128), f32),
            pltpu.VMEM((4, 128), i32),
        ],
    )
    def k(w_h, src_h, tf_h, tot_h, tots, tfb, wb, idxb):
        cc = lax.axis_index("c")
        ss = lax.axis_index("s")
        wid = cc * 16 + ss
        zero16 = jnp.zeros((16,), f32)

        def z_init(i, t):
            for j in range(4):
                tfb[i, pl.ds(j * 16, 16)] = zero16
            return t

        lax.fori_loop(0, 512, z_init, 0)
        pltpu.sync_copy(tfb, tots.at[pl.ds(ss * 626, 512)])
        pltpu.sync_copy(tfb.at[pl.ds(0, 114)],
                        tots.at[pl.ds(ss * 626 + 512, 114)])
        plsc.subcore_barrier()

        def chunk(j, t):
            row0 = wid * 80 + j * 4
            pltpu.sync_copy(tf_h.at[pl.ds(row0 * 128, 512)], tfb)
            pltpu.sync_copy(w_h.at[pl.ds(row0, 4)], wb)
            pltpu.sync_copy(src_h.at[pl.ds(row0, 4)], idxb)

            def pereg(g2, t2):
                w16 = wb[g2 // 8, pl.ds((g2 % 8) * 16, 16)]
                for i in range(16):
                    e = g2 * 16 + i
                    w = w16[i]
                    for j4 in range(4):
                        tfb[e, pl.ds(j4 * 16, 16)] = (
                            tfb[e, pl.ds(j4 * 16, 16)] * w)
                return t2

            lax.fori_loop(0, 32, pereg, 0)
            for b in range(4):
                pltpu.sync_copy(tfb.at[pl.ds(b * 128, 128)],
                                tots.at[idxb.at[b]], add=True)
            return t

        lax.fori_loop(0, 20, chunk, 0)
        plsc.subcore_barrier()
        pltpu.sync_copy(tots.at[pl.ds(ss * 626, 626)],
                        tot_h.at[cc, pl.ds(ss * 626, 626)])

    return k(w2, src2, tf)


def _final_body(tot_ref, x_ref, bs_ref, Wv_ref, bv_ref, Wo_ref, bo_ref,
                Wu_ref, bu_ref, y_ref):
    A = tot_ref[0, :N, :] + tot_ref[1, :N, :]
    combined = (jnp.dot(A, Wv_ref[...], precision="highest")
                + bs_ref[...] * bv_ref[...])
    fused = jnp.maximum(
        jnp.dot(combined, Wo_ref[...], precision="highest") + bo_ref[...], 0.0)
    out = jnp.dot(fused, Wu_ref[...], precision="highest") + bu_ref[...]
    y_ref[...] = x_ref[...] + out


def _final(totals, x, bscalT, Wv, bv, Wout, bout, Wup, bup):
    return pl.pallas_call(
        _final_body,
        out_shape=jax.ShapeDtypeStruct((N, 128), f32),
    )(totals, x, bscalT, Wv, bv, Wout, bout, Wup, bup)


def kernel(x, edge_index, edge_attr, Wdown, bdown, Wtime, btime, Wq, bq,
           Wk, bk, Wv, bv, cluster_emb, Wout, bout, Wup, bup):
    src = edge_index[0].astype(i32)
    src2 = jnp.pad(src, (0, EP - E), constant_values=N).reshape(EPR, 128)
    ea_p = jnp.pad(edge_attr.astype(f32), ((0, EP - E), (0, 0)))

    qwb = _node_dense(x, Wdown, bdown.reshape(1, D), Wq, bq.reshape(1, D),
                      Wk.T, bk.reshape(D, 1))
    qwb_p = jnp.pad(qwb, ((0, 8), (0, 0)))
    qsrc = _sc_gather(qwb_p, src2)

    tf, asg2, attn2, M = _edge_dense(
        ea_p, qsrc, Wtime, btime.reshape(1, D), cluster_emb.T)

    mv = jnp.broadcast_to(M, (1, 128))
    s1, ex2 = _sc_scatter1(attn2, src2, asg2, mv)

    es2 = s1[:, :N * 8, 0].reshape(2, N, NCL).transpose(0, 2, 1)
    cn2 = s1[:, :N * 8, 1].reshape(2, N, NCL).transpose(0, 2, 1)
    scaleT, bscal = _scale(es2, cn2)
    scalef = jnp.pad(scaleT.T.reshape(N * NCL), (0, S1R - N * NCL))

    w2 = _sc_wgather(scalef, ex2, src2, asg2)
    totals = _sc_scatter2(w2, src2, tf)

    return _final(totals, x, bscal.T, Wv, bv.reshape(1, D),
                  Wout, bout.reshape(1, D), Wup, bup.reshape(1, 128))


# speedup vs baseline: 20.8670x; 1.2378x over previous
import functools

import jax
import jax.numpy as jnp
from jax import lax
from jax.experimental import pallas as pl
from jax.experimental.pallas import tpu as pltpu
from jax.experimental.pallas import tpu_sc as plsc

N = 10000
E = 320000
EP = 327680
EPR = EP // 128
D = 64
NCL = 8
S1R = 80128
TR = 10016
NW = 32
EW = EP // NW
SCAL = D ** -0.5

f32 = jnp.float32
i32 = jnp.int32


def _mesh():
    return plsc.VectorSubcoreMesh(
        core_axis_name="c", subcore_axis_name="s", num_cores=2, num_subcores=16)


def _node_dense_body(x_ref, Wd_ref, bd_ref, Wq_ref, bq_ref, WkT_ref, bkc_ref,
                     qwb_ref):
    nf = jnp.maximum(
        jnp.dot(x_ref[...], Wd_ref[...], precision="highest") + bd_ref[...], 0.0)
    q = jnp.dot(nf, Wq_ref[...], precision="highest") + bq_ref[...]
    qW = jnp.dot(q, WkT_ref[...], precision="highest") * SCAL
    qb = jnp.dot(q, bkc_ref[...], precision="highest") * SCAL
    qwb_ref[...] = jnp.concatenate(
        [qW, qb, jnp.zeros((N, 15), f32)], axis=1)


def _node_dense(x, Wd, bd, Wq, bq, WkT, bkc):
    return pl.pallas_call(
        _node_dense_body,
        out_shape=jax.ShapeDtypeStruct((N, 80), f32),
    )(x, Wd, bd, Wq, bq, WkT, bkc)


EB = 6400
GB = E // EB


def _edge_dense_body(ea_ref, Wt_ref, bt_ref, ce_ref, tf_ref, asg_ref):
    tf = jnp.maximum(
        jnp.dot(ea_ref[...], Wt_ref[...], precision="highest") + bt_ref[...], 0.0)
    tf_ref[...] = tf
    simsT = lax.dot_general(ce_ref[...], tf, (((1,), (1,)), ((), ())),
                            precision=lax.Precision.HIGHEST,
                            preferred_element_type=f32)
    best = simsT[0:1, :]
    bi = jnp.zeros((1, EB), i32)
    for c in range(1, NCL):
        sc = simsT[c:c + 1, :]
        m = sc > best
        bi = jnp.where(m, c, bi)
        best = jnp.where(m, sc, best)
    asg_ref[...] = bi.reshape(1, EB // 128, 128)


def _edge_dense(ea, Wt, bt, ce):
    rb = EB // 128
    return pl.pallas_call(
        _edge_dense_body,
        grid=(GB,),
        in_specs=[
            pl.BlockSpec((EB, 16), lambda i: (i, 0)),
            pl.BlockSpec((16, D), lambda i: (0, 0)),
            pl.BlockSpec((1, D), lambda i: (0, 0)),
            pl.BlockSpec((NCL, D), lambda i: (0, 0)),
        ],
        out_specs=[
            pl.BlockSpec((EB, D), lambda i: (i, 0)),
            pl.BlockSpec((1, rb, 128), lambda i: (i, 0, 0)),
        ],
        out_shape=[
            jax.ShapeDtypeStruct((EP, D), f32),
            jax.ShapeDtypeStruct((GB, rb, 128), i32),
        ],
    )(ea, Wt, bt, ce)


def _sc_gatherdot(qwb_p, src2, tf):
    @functools.partial(
        pl.kernel,
        out_type=(
            jax.ShapeDtypeStruct((EPR, 128), f32),
            jax.ShapeDtypeStruct((NW, 128), f32),
        ),
        mesh=_mesh(),
        compiler_params=pltpu.CompilerParams(use_tc_tiling_on_sc=False, needs_layout_passes=False),
        scratch_types=[
            pltpu.VMEM((4, 128), i32),
            pltpu.VMEM((512, 80), f32),
            pltpu.VMEM((512, D), f32),
            pltpu.VMEM((4, 128), f32),
            pltpu.VMEM((1, 128), f32),
            pltpu.SemaphoreType.DMA,
        ],
    )
    def k(qwb_h, src_h, tf_h, attn_h, mp_h, sidx, qrows, tfb, ab, mbuf, sem):
        wid = lax.axis_index("c") * 16 + lax.axis_index("s")
        lanes = lax.iota(i32, 16)

        def chunk(j, mx):
            row0 = wid * 80 + j * 4
            pltpu.sync_copy(src_h.at[pl.ds(row0, 4)], sidx)
            descs = [
                pltpu.async_copy(qwb_h.at[sidx.at[b]],
                                 qrows.at[pl.ds(b * 128, 128)], sem)
                for b in range(4)
            ]
            pltpu.sync_copy(tf_h.at[pl.ds(row0 * 128, 512)], tfb)
            for dsc in descs:
                dsc.wait()

            def grp(g, mx2):
                b, off = g // 8, (g % 8) * 16
                e16 = g * 16 + lanes
                acc = jnp.zeros((16,), f32)
                for d in range(D):
                    dd = jnp.full((16,), d, i32)
                    acc = acc + (plsc.load_gather(qrows, [e16, dd]) *
                                 plsc.load_gather(tfb, [e16, dd]))
                qb = plsc.load_gather(qrows, [e16, jnp.full((16,), D, i32)])
                attn = acc + qb
                eidx = (row0 + b) * 128 + off + lanes
                msk = eidx < E
                attn = jnp.where(msk, attn, 0.0)
                ab[b, pl.ds(off, 16)] = attn
                return jnp.maximum(mx2, jnp.where(msk, attn, -1e30))

            mx = lax.fori_loop(0, 32, grp, mx)
            pltpu.sync_copy(ab, attn_h.at[pl.ds(row0, 4)])
            return mx

        mx = lax.fori_loop(0, 20, chunk, jnp.full((16,), -1e30, f32))
        mbuf[0, pl.ds(0, 16)] = mx
        pltpu.sync_copy(mbuf, mp_h.at[pl.ds(wid, 1)])

    return k(qwb_p, src2, tf)


def _sc_scatter1(attn2, src2, asg2, mpart):
    @functools.partial(
        pl.kernel,
        out_type=(
            jax.ShapeDtypeStruct((2, S1R, 16), f32),
            jax.ShapeDtypeStruct((EPR, 128), f32),
        ),
        mesh=_mesh(),
        compiler_params=pltpu.CompilerParams(use_tc_tiling_on_sc=False, needs_layout_passes=False),
        scratch_types=[
            pltpu.VMEM_SHARED((S1R, 16), f32),
            pltpu.VMEM((8, 128), f32),
            pltpu.VMEM((8, 128), i32),
            pltpu.VMEM((8, 128), i32),
            pltpu.VMEM((8, 128), f32),
            pltpu.VMEM((8, 128), i32),
            pltpu.VMEM((1024, 16), f32),
            pltpu.VMEM((NW, 128), f32),
        ],
    )
    def k(attn_h, src_h, asg_h, mp_h, s1_h, ex_h,
          s1s, abuf, sbuf, cbuf, exbuf, idxb, rowb, mb):
        cc = lax.axis_index("c")
        ss = lax.axis_index("s")
        wid = cc * 16 + ss
        zero16 = jnp.zeros((16,), f32)
        tmpl = jnp.where(lax.iota(i32, 16) == 1, 1.0, 0.0).astype(f32)

        def zb_init(i, t):
            rowb[i, :] = zero16
            return t

        lax.fori_loop(0, 1024, zb_init, 0)
        for p in range(4):
            pltpu.sync_copy(rowb, s1s.at[pl.ds(ss * 5008 + p * 1024, 1024)])
        pltpu.sync_copy(rowb.at[pl.ds(0, 912)],
                        s1s.at[pl.ds(ss * 5008 + 4096, 912)])

        def rb_init(i, t):
            rowb[i, :] = tmpl
            return t

        lax.fori_loop(0, 1024, rb_init, 0)
        pltpu.sync_copy(mp_h, mb)
        plsc.subcore_barrier()

        def mred(r, mv):
            return jnp.maximum(mv, mb[r, pl.ds(0, 16)])

        mvec16 = lax.fori_loop(0, NW, mred, jnp.full((16,), -1e30, f32))
        M = jnp.max(mvec16)
        lanes = lax.iota(i32, 16)
        zlanes = jnp.zeros((16,), i32)

        def chunk(j, t):
            row0 = wid * 80 + j * 8
            pltpu.sync_copy(attn_h.at[pl.ds(row0, 8)], abuf)
            pltpu.sync_copy(src_h.at[pl.ds(row0, 8)], sbuf)
            pltpu.sync_copy(asg_h.at[pl.ds(row0, 8)], cbuf)
            for g in range(64):
                b, off = g // 8, (g % 8) * 16
                eidx = (row0 + b) * 128 + off + lanes
                msk = eidx < E
                ex = jnp.where(msk, jnp.exp(abuf[b, pl.ds(off, 16)] - M), 0.0)
                exbuf[b, pl.ds(off, 16)] = ex
                idx = sbuf[b, pl.ds(off, 16)] * 8 + cbuf[b, pl.ds(off, 16)]
                idxb[b, pl.ds(off, 16)] = jnp.where(msk, idx, N * NCL)
                plsc.store_scatter(rowb, [g * 16 + lanes, zlanes], ex)
            pltpu.sync_copy(exbuf, ex_h.at[pl.ds(row0, 8)])
            for b in range(8):
                pltpu.sync_copy(rowb.at[pl.ds(b * 128, 128)],
                                s1s.at[idxb.at[b]], add=True)
            return t

        lax.fori_loop(0, 10, chunk, 0)
        plsc.subcore_barrier()
        pltpu.sync_copy(s1s.at[pl.ds(ss * 5008, 5008)],
                        s1_h.at[cc, pl.ds(ss * 5008, 5008)])

    return k(attn2, src2, asg2, mpart)


def _scale_body(es_ref, cn_ref, scaleT_ref, bscal_ref):
    es = es_ref[0] + es_ref[1]
    cn = cn_ref[0] + cn_ref[1]
    g = (jnp.sum(cn, axis=1, keepdims=True) > 0).astype(f32)
    nne = jnp.sum(g)
    mc = jnp.maximum(cn, 1.0)
    scaleT_ref[...] = g / (jnp.where(es > 0, es, 1.0) * mc * nne)
    bscal_ref[...] = jnp.sum(
        g * (cn > 0).astype(f32) / (mc * nne), axis=0, keepdims=True)


def _scale(es2, cn2):
    return pl.pallas_call(
        _scale_body,
        out_shape=[
            jax.ShapeDtypeStruct((NCL, N), f32),
            jax.ShapeDtypeStruct((1, N), f32),
        ],
    )(es2, cn2)


def _sc_wgather(scalef, ex2, src2, asg2):
    @functools.partial(
        pl.kernel,
        out_type=jax.ShapeDtypeStruct((EPR, 128), f32),
        mesh=_mesh(),
        compiler_params=pltpu.CompilerParams(use_tc_tiling_on_sc=False, needs_layout_passes=False),
        scratch_types=[
            pltpu.VMEM((S1R,), f32),
            pltpu.VMEM((16, 128), f32),
            pltpu.VMEM((16, 128), i32),
            pltpu.VMEM((16, 128), i32),
            pltpu.VMEM((16, 128), f32),
        ],
    )
    def k(scale_h, ex_h, src_h, asg_h, w_h, scv, exb, sb, cb, wb):
        wid = lax.axis_index("c") * 16 + lax.axis_index("s")
        pltpu.sync_copy(scale_h, scv)

        def chunk(j, t):
            row0 = wid * 80 + j * 16
            pltpu.sync_copy(ex_h.at[pl.ds(row0, 16)], exb)
            pltpu.sync_copy(src_h.at[pl.ds(row0, 16)], sb)
            pltpu.sync_copy(asg_h.at[pl.ds(row0, 16)], cb)
            lanes = lax.iota(i32, 16)
            for g in range(128):
                b, off = g // 8, (g % 8) * 16
                eidx = (row0 + b) * 128 + off + lanes
                idx = sb[b, pl.ds(off, 16)] * 8 + cb[b, pl.ds(off, 16)]
                idx = jnp.where(eidx < E, idx, N * NCL)
                sc16 = plsc.load_gather(scv, [idx])
                wb[b, pl.ds(off, 16)] = exb[b, pl.ds(off, 16)] * sc16
            pltpu.sync_copy(wb, w_h.at[pl.ds(row0, 16)])
            return t

        lax.fori_loop(0, 5, chunk, 0)

    return k(scalef, ex2, src2, asg2)


def _sc_scatter2(w2, src2, tf):
    @functools.partial(
        pl.kernel,
        out_type=jax.ShapeDtypeStruct((2, TR, D), f32),
        mesh=_mesh(),
        compiler_params=pltpu.CompilerParams(use_tc_tiling_on_sc=False, needs_layout_passes=False),
        scratch_types=[
            pltpu.VMEM_SHARED((TR, D), f32),
            pltpu.VMEM((512, D), f32),
            pltpu.VMEM((4, 128), f32),
            pltpu.VMEM((4, 128), i32),
        ],
    )
    def k(w_h, src_h, tf_h, tot_h, tots, tfb, wb, idxb):
        cc = lax.axis_index("c")
        ss = lax.axis_index("s")
        wid = cc * 16 + ss
        zero16 = jnp.zeros((16,), f32)

        def z_init(i, t):
            for j in range(4):
                tfb[i, pl.ds(j * 16, 16)] = zero16
            return t

        lax.fori_loop(0, 512, z_init, 0)
        pltpu.sync_copy(tfb, tots.at[pl.ds(ss * 626, 512)])
        pltpu.sync_copy(tfb.at[pl.ds(0, 114)],
                        tots.at[pl.ds(ss * 626 + 512, 114)])
        plsc.subcore_barrier()

        def chunk(j, t):
            row0 = wid * 80 + j * 4
            pltpu.sync_copy(tf_h.at[pl.ds(row0 * 128, 512)], tfb)
            pltpu.sync_copy(w_h.at[pl.ds(row0, 4)], wb)
            pltpu.sync_copy(src_h.at[pl.ds(row0, 4)], idxb)

            def pereg(g2, t2):
                w16 = wb[g2 // 8, pl.ds((g2 % 8) * 16, 16)]
                for i in range(16):
                    e = g2 * 16 + i
                    w = w16[i]
                    for j4 in range(4):
                        tfb[e, pl.ds(j4 * 16, 16)] = (
                            tfb[e, pl.ds(j4 * 16, 16)] * w)
                return t2

            lax.fori_loop(0, 32, pereg, 0)
            for b in range(4):
                pltpu.sync_copy(tfb.at[pl.ds(b * 128, 128)],
                                tots.at[idxb.at[b]], add=True)
            return t

        lax.fori_loop(0, 20, chunk, 0)
        plsc.subcore_barrier()
        pltpu.sync_copy(tots.at[pl.ds(ss * 626, 626)],
                        tot_h.at[cc, pl.ds(ss * 626, 626)])

    return k(w2, src2, tf)


def _final_body(tot_ref, x_ref, bs_ref, Wv_ref, bv_ref, Wo_ref, bo_ref,
                Wu_ref, bu_ref, y_ref):
    A = tot_ref[0, :N, :] + tot_ref[1, :N, :]
    combined = (jnp.dot(A, Wv_ref[...], precision="highest")
                + bs_ref[...] * bv_ref[...])
    fused = jnp.maximum(
        jnp.dot(combined, Wo_ref[...], precision="highest") + bo_ref[...], 0.0)
    out = jnp.dot(fused, Wu_ref[...], precision="highest") + bu_ref[...]
    y_ref[...] = x_ref[...] + out


def _final(totals, x, bscalT, Wv, bv, Wout, bout, Wup, bup):
    return pl.pallas_call(
        _final_body,
        out_shape=jax.ShapeDtypeStruct((N, 128), f32),
    )(totals, x, bscalT, Wv, bv, Wout, bout, Wup, bup)


def kernel(x, edge_index, edge_attr, Wdown, bdown, Wtime, btime, Wq, bq,
           Wk, bk, Wv, bv, cluster_emb, Wout, bout, Wup, bup):
    src = edge_index[0].astype(i32)
    src2 = jnp.pad(src, (0, EP - E), constant_values=N).reshape(EPR, 128)

    qwb = _node_dense(x, Wdown, bdown.reshape(1, D), Wq, bq.reshape(1, D),
                      Wk.T, bk.reshape(D, 1))
    qwb_p = jnp.pad(qwb, ((0, 8), (0, 0)))

    tf, asg3 = _edge_dense(edge_attr, Wtime, btime.reshape(1, D), cluster_emb)
    asg2 = jnp.concatenate(
        [asg3.reshape(GB * (EB // 128), 128),
         jnp.zeros((EPR - GB * (EB // 128), 128), i32)], axis=0)
    attn2, mpart = _sc_gatherdot(qwb_p, src2, tf)

    s1, ex2 = _sc_scatter1(attn2, src2, asg2, mpart)

    es2 = s1[:, :N * 8, 0].reshape(2, N, NCL).transpose(0, 2, 1)
    cn2 = s1[:, :N * 8, 1].reshape(2, N, NCL).transpose(0, 2, 1)
    scaleT, bscal = _scale(es2, cn2)
    scalef = jnp.pad(scaleT.T.reshape(N * NCL), (0, S1R - N * NCL))

    w2 = _sc_wgather(scalef, ex2, src2, asg2)
    totals = _sc_scatter2(w2, src2, tf)

    return _final(totals, x, bscal.T, Wv, bv.reshape(1, D),
                  Wout, bout.reshape(1, D), Wup, bup.reshape(1, 128))


# speedup vs baseline: 22.5434x; 1.0803x over previous
import functools

import jax
import jax.numpy as jnp
from jax import lax
from jax.experimental import pallas as pl
from jax.experimental.pallas import tpu as pltpu
from jax.experimental.pallas import tpu_sc as plsc

N = 10000
E = 320000
EP = 327680
EPR = EP // 128
D = 64
NCL = 8
S1R = 80128
TR = 10016
NW = 32
EW = EP // NW
SCAL = D ** -0.5

f32 = jnp.float32
i32 = jnp.int32


def _mesh():
    return plsc.VectorSubcoreMesh(
        core_axis_name="c", subcore_axis_name="s", num_cores=2, num_subcores=16)


def _node_dense_body(x_ref, Wd_ref, bd_ref, Wq_ref, bq_ref, WkT_ref, bkc_ref,
                     qwb_ref):
    nf = jnp.maximum(
        jnp.dot(x_ref[...], Wd_ref[...], precision="highest") + bd_ref[...], 0.0)
    q = jnp.dot(nf, Wq_ref[...], precision="highest") + bq_ref[...]
    qW = jnp.dot(q, WkT_ref[...], precision="highest") * SCAL
    qb = jnp.dot(q, bkc_ref[...], precision="highest") * SCAL
    qwb_ref[...] = jnp.concatenate(
        [qW, qb, jnp.zeros((N, 15), f32)], axis=1)


def _node_dense(x, Wd, bd, Wq, bq, WkT, bkc):
    return pl.pallas_call(
        _node_dense_body,
        out_shape=jax.ShapeDtypeStruct((N, 80), f32),
    )(x, Wd, bd, Wq, bq, WkT, bkc)


EB = 6400
GB = E // EB


def _edge_dense_body(ea_ref, Wt_ref, bt_ref, ce_ref, tf_ref, asg_ref):
    tf = jnp.maximum(
        jnp.dot(ea_ref[...], Wt_ref[...], precision="highest") + bt_ref[...], 0.0)
    tf_ref[...] = tf
    simsT = lax.dot_general(ce_ref[...], tf, (((1,), (1,)), ((), ())),
                            precision=lax.Precision.HIGHEST,
                            preferred_element_type=f32)
    best = simsT[0:1, :]
    bi = jnp.zeros((1, EB), i32)
    for c in range(1, NCL):
        sc = simsT[c:c + 1, :]
        m = sc > best
        bi = jnp.where(m, c, bi)
        best = jnp.where(m, sc, best)
    asg_ref[...] = bi.reshape(1, EB // 128, 128)


def _edge_dense(ea, Wt, bt, ce):
    rb = EB // 128
    return pl.pallas_call(
        _edge_dense_body,
        grid=(GB,),
        in_specs=[
            pl.BlockSpec((EB, 16), lambda i: (i, 0)),
            pl.BlockSpec((16, D), lambda i: (0, 0)),
            pl.BlockSpec((1, D), lambda i: (0, 0)),
            pl.BlockSpec((NCL, D), lambda i: (0, 0)),
        ],
        out_specs=[
            pl.BlockSpec((EB, D), lambda i: (i, 0)),
            pl.BlockSpec((1, rb, 128), lambda i: (i, 0, 0)),
        ],
        out_shape=[
            jax.ShapeDtypeStruct((EP, D), f32),
            jax.ShapeDtypeStruct((GB, rb, 128), i32),
        ],
    )(ea, Wt, bt, ce)


def _sc_gatherdot(qwb_p, src2, tf):
    @functools.partial(
        pl.kernel,
        out_type=(
            jax.ShapeDtypeStruct((EPR, 128), f32),
            jax.ShapeDtypeStruct((NW, 128), f32),
        ),
        mesh=_mesh(),
        compiler_params=pltpu.CompilerParams(use_tc_tiling_on_sc=False, needs_layout_passes=False),
        scratch_types=[
            pltpu.VMEM((4, 128), i32),
            pltpu.VMEM((512, 80), f32),
            pltpu.VMEM((512, D), f32),
            pltpu.VMEM((4, 128), f32),
            pltpu.VMEM((1, 128), f32),
            pltpu.SemaphoreType.DMA,
        ],
    )
    def k(qwb_h, src_h, tf_h, attn_h, mp_h, sidx, qrows, tfb, ab, mbuf, sem):
        wid = lax.axis_index("c") * 16 + lax.axis_index("s")
        lanes = lax.iota(i32, 16)
        lane0 = lanes == 0

        def chunk(j, mx):
            row0 = wid * 80 + j * 4
            pltpu.sync_copy(src_h.at[pl.ds(row0, 4)], sidx)
            descs = [
                pltpu.async_copy(qwb_h.at[sidx.at[b]],
                                 qrows.at[pl.ds(b * 128, 128)], sem)
                for b in range(4)
            ]
            pltpu.sync_copy(tf_h.at[pl.ds(row0 * 128, 512)], tfb)
            for dsc in descs:
                dsc.wait()

            def grp(g, mx2):
                b, off = g // 8, (g % 8) * 16
                bb = jnp.full((16,), b, i32)
                for i in range(16):
                    e = g * 16 + i
                    acc = qrows[e, pl.ds(0, 16)] * tfb[e, pl.ds(0, 16)]
                    for j4 in range(1, 4):
                        acc = acc + (qrows[e, pl.ds(j4 * 16, 16)] *
                                     tfb[e, pl.ds(j4 * 16, 16)])
                    s = jnp.sum(acc) + qrows[e, pl.ds(D, 16)][0]
                    plsc.store_scatter(
                        ab, [bb, jnp.full((16,), off + i, i32)],
                        jnp.full((16,), s, f32), mask=lane0)
                eidx = (row0 + b) * 128 + off + lanes
                msk = eidx < E
                a16 = jnp.where(msk, ab[b, pl.ds(off, 16)], 0.0)
                ab[b, pl.ds(off, 16)] = a16
                return jnp.maximum(mx2, jnp.where(msk, a16, -1e30))

            mx = lax.fori_loop(0, 32, grp, mx)
            pltpu.sync_copy(ab, attn_h.at[pl.ds(row0, 4)])
            return mx

        mx = lax.fori_loop(0, 20, chunk, jnp.full((16,), -1e30, f32))
        mbuf[0, pl.ds(0, 16)] = mx
        pltpu.sync_copy(mbuf, mp_h.at[pl.ds(wid, 1)])

    return k(qwb_p, src2, tf)


def _sc_scatter1(attn2, src2, asg2, mpart):
    @functools.partial(
        pl.kernel,
        out_type=(
            jax.ShapeDtypeStruct((2, S1R, 16), f32),
            jax.ShapeDtypeStruct((EPR, 128), f32),
        ),
        mesh=_mesh(),
        compiler_params=pltpu.CompilerParams(use_tc_tiling_on_sc=False, needs_layout_passes=False),
        scratch_types=[
            pltpu.VMEM_SHARED((S1R, 16), f32),
            pltpu.VMEM((8, 128), f32),
            pltpu.VMEM((8, 128), i32),
            pltpu.VMEM((8, 128), i32),
            pltpu.VMEM((8, 128), f32),
            pltpu.VMEM((8, 128), i32),
            pltpu.VMEM((1024, 16), f32),
            pltpu.VMEM((NW, 128), f32),
            pltpu.SemaphoreType.DMA,
        ],
    )
    def k(attn_h, src_h, asg_h, mp_h, s1_h, ex_h,
          s1s, abuf, sbuf, cbuf, exbuf, idxb, rowb, mb, sem):
        cc = lax.axis_index("c")
        ss = lax.axis_index("s")
        wid = cc * 16 + ss
        zero16 = jnp.zeros((16,), f32)
        tmpl = jnp.where(lax.iota(i32, 16) == 1, 1.0, 0.0).astype(f32)

        def zb_init(i, t):
            rowb[i, :] = zero16
            return t

        lax.fori_loop(0, 1024, zb_init, 0)
        for p in range(4):
            pltpu.sync_copy(rowb, s1s.at[pl.ds(ss * 5008 + p * 1024, 1024)])
        pltpu.sync_copy(rowb.at[pl.ds(0, 912)],
                        s1s.at[pl.ds(ss * 5008 + 4096, 912)])

        def rb_init(i, t):
            rowb[i, :] = tmpl
            return t

        lax.fori_loop(0, 1024, rb_init, 0)
        pltpu.sync_copy(mp_h, mb)
        plsc.subcore_barrier()

        def mred(r, mv):
            return jnp.maximum(mv, mb[r, pl.ds(0, 16)])

        mvec16 = lax.fori_loop(0, NW, mred, jnp.full((16,), -1e30, f32))
        M = jnp.max(mvec16)
        lanes = lax.iota(i32, 16)
        zlanes = jnp.zeros((16,), i32)

        def chunk(j, t):
            row0 = wid * 80 + j * 8
            pltpu.sync_copy(attn_h.at[pl.ds(row0, 8)], abuf)
            pltpu.sync_copy(src_h.at[pl.ds(row0, 8)], sbuf)
            pltpu.sync_copy(asg_h.at[pl.ds(row0, 8)], cbuf)
            for g in range(64):
                b, off = g // 8, (g % 8) * 16
                eidx = (row0 + b) * 128 + off + lanes
                msk = eidx < E
                ex = jnp.where(msk, jnp.exp(abuf[b, pl.ds(off, 16)] - M), 0.0)
                exbuf[b, pl.ds(off, 16)] = ex
                idx = sbuf[b, pl.ds(off, 16)] * 8 + cbuf[b, pl.ds(off, 16)]
                idxb[b, pl.ds(off, 16)] = jnp.where(msk, idx, N * NCL)
                plsc.store_scatter(rowb, [g * 16 + lanes, zlanes], ex)
            pltpu.sync_copy(exbuf, ex_h.at[pl.ds(row0, 8)])
            descs = [
                pltpu.async_copy(rowb.at[pl.ds(b * 128, 128)],
                                 s1s.at[idxb.at[b]], sem, add=True)
                for b in range(8)
            ]
            for dsc in descs:
                dsc.wait()
            return t

        lax.fori_loop(0, 10, chunk, 0)
        plsc.subcore_barrier()
        pltpu.sync_copy(s1s.at[pl.ds(ss * 5008, 5008)],
                        s1_h.at[cc, pl.ds(ss * 5008, 5008)])

    return k(attn2, src2, asg2, mpart)


def _scale_body(es_ref, cn_ref, scaleT_ref, bscal_ref):
    es = es_ref[0] + es_ref[1]
    cn = cn_ref[0] + cn_ref[1]
    g = (jnp.sum(cn, axis=1, keepdims=True) > 0).astype(f32)
    nne = jnp.sum(g)
    mc = jnp.maximum(cn, 1.0)
    scaleT_ref[...] = g / (jnp.where(es > 0, es, 1.0) * mc * nne)
    bscal_ref[...] = jnp.sum(
        g * (cn > 0).astype(f32) / (mc * nne), axis=0, keepdims=True)


def _scale(es2, cn2):
    return pl.pallas_call(
        _scale_body,
        out_shape=[
            jax.ShapeDtypeStruct((NCL, N), f32),
            jax.ShapeDtypeStruct((1, N), f32),
        ],
    )(es2, cn2)


def _sc_wgather(scalef, ex2, src2, asg2):
    @functools.partial(
        pl.kernel,
        out_type=jax.ShapeDtypeStruct((EPR, 128), f32),
        mesh=_mesh(),
        compiler_params=pltpu.CompilerParams(use_tc_tiling_on_sc=False, needs_layout_passes=False),
        scratch_types=[
            pltpu.VMEM((S1R,), f32),
            pltpu.VMEM((16, 128), f32),
            pltpu.VMEM((16, 128), i32),
            pltpu.VMEM((16, 128), i32),
            pltpu.VMEM((16, 128), f32),
        ],
    )
    def k(scale_h, ex_h, src_h, asg_h, w_h, scv, exb, sb, cb, wb):
        wid = lax.axis_index("c") * 16 + lax.axis_index("s")
        pltpu.sync_copy(scale_h, scv)

        def chunk(j, t):
            row0 = wid * 80 + j * 16
            pltpu.sync_copy(ex_h.at[pl.ds(row0, 16)], exb)
            pltpu.sync_copy(src_h.at[pl.ds(row0, 16)], sb)
            pltpu.sync_copy(asg_h.at[pl.ds(row0, 16)], cb)
            lanes = lax.iota(i32, 16)
            for g in range(128):
                b, off = g // 8, (g % 8) * 16
                eidx = (row0 + b) * 128 + off + lanes
                idx = sb[b, pl.ds(off, 16)] * 8 + cb[b, pl.ds(off, 16)]
                idx = jnp.where(eidx < E, idx, N * NCL)
                sc16 = plsc.load_gather(scv, [idx])
                wb[b, pl.ds(off, 16)] = exb[b, pl.ds(off, 16)] * sc16
            pltpu.sync_copy(wb, w_h.at[pl.ds(row0, 16)])
            return t

        lax.fori_loop(0, 5, chunk, 0)

    return k(scalef, ex2, src2, asg2)


def _sc_scatter2(w2, src2, tf):
    @functools.partial(
        pl.kernel,
        out_type=jax.ShapeDtypeStruct((2, TR, D), f32),
        mesh=_mesh(),
        compiler_params=pltpu.CompilerParams(use_tc_tiling_on_sc=False, needs_layout_passes=False),
        scratch_types=[
            pltpu.VMEM_SHARED((TR, D), f32),
            pltpu.VMEM((512, D), f32),
            pltpu.VMEM((4, 128), f32),
            pltpu.VMEM((4, 128), i32),
            pltpu.SemaphoreType.DMA,
        ],
    )
    def k(w_h, src_h, tf_h, tot_h, tots, tfb, wb, idxb, sem):
        cc = lax.axis_index("c")
        ss = lax.axis_index("s")
        wid = cc * 16 + ss
        zero16 = jnp.zeros((16,), f32)

        def z_init(i, t):
            for j in range(4):
                tfb[i, pl.ds(j * 16, 16)] = zero16
            return t

        lax.fori_loop(0, 512, z_init, 0)
        pltpu.sync_copy(tfb, tots.at[pl.ds(ss * 626, 512)])
        pltpu.sync_copy(tfb.at[pl.ds(0, 114)],
                        tots.at[pl.ds(ss * 626 + 512, 114)])
        plsc.subcore_barrier()

        def chunk(j, t):
            row0 = wid * 80 + j * 4
            pltpu.sync_copy(tf_h.at[pl.ds(row0 * 128, 512)], tfb)
            pltpu.sync_copy(w_h.at[pl.ds(row0, 4)], wb)
            pltpu.sync_copy(src_h.at[pl.ds(row0, 4)], idxb)

            def pereg(g2, t2):
                w16 = wb[g2 // 8, pl.ds((g2 % 8) * 16, 16)]
                for i in range(16):
                    e = g2 * 16 + i
                    w = w16[i]
                    for j4 in range(4):
                        tfb[e, pl.ds(j4 * 16, 16)] = (
                            tfb[e, pl.ds(j4 * 16, 16)] * w)
                return t2

            lax.fori_loop(0, 32, pereg, 0)
            descs = [
                pltpu.async_copy(tfb.at[pl.ds(b * 128, 128)],
                                 tots.at[idxb.at[b]], sem, add=True)
                for b in range(4)
            ]
            for dsc in descs:
                dsc.wait()
            return t

        lax.fori_loop(0, 20, chunk, 0)
        plsc.subcore_barrier()
        pltpu.sync_copy(tots.at[pl.ds(ss * 626, 626)],
                        tot_h.at[cc, pl.ds(ss * 626, 626)])

    return k(w2, src2, tf)


def _final_body(tot_ref, x_ref, bs_ref, Wv_ref, bv_ref, Wo_ref, bo_ref,
                Wu_ref, bu_ref, y_ref):
    A = tot_ref[0, :N, :] + tot_ref[1, :N, :]
    combined = (jnp.dot(A, Wv_ref[...], precision="highest")
                + bs_ref[...] * bv_ref[...])
    fused = jnp.maximum(
        jnp.dot(combined, Wo_ref[...], precision="highest") + bo_ref[...], 0.0)
    out = jnp.dot(fused, Wu_ref[...], precision="highest") + bu_ref[...]
    y_ref[...] = x_ref[...] + out


def _final(totals, x, bscalT, Wv, bv, Wout, bout, Wup, bup):
    return pl.pallas_call(
        _final_body,
        out_shape=jax.ShapeDtypeStruct((N, 128), f32),
    )(totals, x, bscalT, Wv, bv, Wout, bout, Wup, bup)


def kernel(x, edge_index, edge_attr, Wdown, bdown, Wtime, btime, Wq, bq,
           Wk, bk, Wv, bv, cluster_emb, Wout, bout, Wup, bup):
    src = edge_index[0].astype(i32)
    src2 = jnp.concatenate(
        [src.reshape(E // 128, 128), jnp.full((EPR - E // 128, 128), N, i32)],
        axis=0)

    qwb = _node_dense(x, Wdown, bdown.reshape(1, D), Wq, bq.reshape(1, D),
                      Wk.T, bk.reshape(D, 1))
    qwb_p = jnp.pad(qwb, ((0, 8), (0, 0)))

    tf, asg3 = _edge_dense(edge_attr, Wtime, btime.reshape(1, D), cluster_emb)
    asg2 = jnp.concatenate(
        [asg3.reshape(GB * (EB // 128), 128),
         jnp.zeros((EPR - GB * (EB // 128), 128), i32)], axis=0)
    attn2, mpart = _sc_gatherdot(qwb_p, src2, tf)

    s1, ex2 = _sc_scatter1(attn2, src2, asg2, mpart)

    es2 = s1[:, :N * 8, 0].reshape(2, N, NCL).transpose(0, 2, 1)
    cn2 = s1[:, :N * 8, 1].reshape(2, N, NCL).transpose(0, 2, 1)
    scaleT, bscal = _scale(es2, cn2)
    scalef = jnp.pad(scaleT.T.reshape(N * NCL), (0, S1R - N * NCL))

    w2 = _sc_wgather(scalef, ex2, src2, asg2)
    totals = _sc_scatter2(w2, src2, tf)

    return _final(totals, x, bscal.T, Wv, bv.reshape(1, D),
                  Wout, bout.reshape(1, D), Wup, bup.reshape(1, 128))


# speedup vs baseline: 27.1732x; 1.2054x over previous
import functools

import jax
import jax.numpy as jnp
from jax import lax
from jax.experimental import pallas as pl
from jax.experimental.pallas import tpu as pltpu
from jax.experimental.pallas import tpu_sc as plsc

N = 10000
E = 320000
EP = 327680
EPR = EP // 128
D = 64
NCL = 8
S1R = 80128
TR = 10016
NW = 32
EW = EP // NW
SCAL = D ** -0.5

f32 = jnp.float32
i32 = jnp.int32


def _mesh():
    return plsc.VectorSubcoreMesh(
        core_axis_name="c", subcore_axis_name="s", num_cores=2, num_subcores=16)


def _node_dense_body(x_ref, Wd_ref, bd_ref, Wq_ref, bq_ref, WkT_ref, bkc_ref,
                     qwb_ref):
    nf = jnp.maximum(
        jnp.dot(x_ref[...], Wd_ref[...], precision="highest") + bd_ref[...], 0.0)
    q = jnp.dot(nf, Wq_ref[...], precision="highest") + bq_ref[...]
    qW = jnp.dot(q, WkT_ref[...], precision="highest") * SCAL
    qb = jnp.dot(q, bkc_ref[...], precision="highest") * SCAL
    qwb_ref[...] = jnp.concatenate(
        [qW, qb, jnp.zeros((N, 15), f32)], axis=1)


def _node_dense(x, Wd, bd, Wq, bq, WkT, bkc):
    return pl.pallas_call(
        _node_dense_body,
        out_shape=jax.ShapeDtypeStruct((N, 80), f32),
    )(x, Wd, bd, Wq, bq, WkT, bkc)


EB = 6400
GB = E // EB


def _edge_dense_body(ea_ref, Wt_ref, bt_ref, ce_ref, tf_ref, asg_ref):
    tf = jnp.maximum(
        jnp.dot(ea_ref[...], Wt_ref[...], precision="float32") + bt_ref[...], 0.0)
    tf_ref[...] = tf
    simsT = lax.dot_general(ce_ref[...], tf, (((1,), (1,)), ((), ())),
                            precision=lax.Precision.DEFAULT,
                            preferred_element_type=f32)
    best = simsT[0:1, :]
    bi = jnp.zeros((1, EB), i32)
    for c in range(1, NCL):
        sc = simsT[c:c + 1, :]
        m = sc > best
        bi = jnp.where(m, c, bi)
        best = jnp.where(m, sc, best)
    asg_ref[...] = bi.reshape(1, EB // 128, 128)


def _edge_dense(ea, Wt, bt, ce):
    rb = EB // 128
    return pl.pallas_call(
        _edge_dense_body,
        grid=(GB,),
        in_specs=[
            pl.BlockSpec((EB, 16), lambda i: (i, 0)),
            pl.BlockSpec((16, D), lambda i: (0, 0)),
            pl.BlockSpec((1, D), lambda i: (0, 0)),
            pl.BlockSpec((NCL, D), lambda i: (0, 0)),
        ],
        out_specs=[
            pl.BlockSpec((EB, D), lambda i: (i, 0)),
            pl.BlockSpec((1, rb, 128), lambda i: (i, 0, 0)),
        ],
        out_shape=[
            jax.ShapeDtypeStruct((EP, D), f32),
            jax.ShapeDtypeStruct((GB, rb, 128), i32),
        ],
    )(ea, Wt, bt, ce)


def _sc_gatherdot(qwb_p, src2, tf):
    @functools.partial(
        pl.kernel,
        out_type=(
            jax.ShapeDtypeStruct((EPR, 128), f32),
            jax.ShapeDtypeStruct((NW, 128), f32),
        ),
        mesh=_mesh(),
        compiler_params=pltpu.CompilerParams(use_tc_tiling_on_sc=False, needs_layout_passes=False),
        scratch_types=[
            pltpu.VMEM((4, 128), i32),
            pltpu.VMEM((512, 80), f32),
            pltpu.VMEM((512, D), f32),
            pltpu.VMEM((4, 128), f32),
            pltpu.VMEM((1, 128), f32),
            pltpu.SemaphoreType.DMA,
        ],
    )
    def k(qwb_h, src_h, tf_h, attn_h, mp_h, sidx, qrows, tfb, ab, mbuf, sem):
        wid = lax.axis_index("c") * 16 + lax.axis_index("s")
        lanes = lax.iota(i32, 16)
        lane0 = lanes == 0

        def chunk(j, mx):
            row0 = wid * 80 + j * 4
            pltpu.sync_copy(src_h.at[pl.ds(row0, 4)], sidx)
            descs = [
                pltpu.async_copy(qwb_h.at[sidx.at[b]],
                                 qrows.at[pl.ds(b * 128, 128)], sem)
                for b in range(4)
            ]
            pltpu.sync_copy(tf_h.at[pl.ds(row0 * 128, 512)], tfb)
            for dsc in descs:
                dsc.wait()

            def grp(g, mx2):
                b, off = g // 8, (g % 8) * 16
                bb = jnp.full((16,), b, i32)
                for i in range(16):
                    e = g * 16 + i
                    acc = qrows[e, pl.ds(0, 16)] * tfb[e, pl.ds(0, 16)]
                    for j4 in range(1, 4):
                        acc = acc + (qrows[e, pl.ds(j4 * 16, 16)] *
                                     tfb[e, pl.ds(j4 * 16, 16)])
                    s = jnp.sum(acc) + qrows[e, pl.ds(D, 16)][0]
                    plsc.store_scatter(
                        ab, [bb, jnp.full((16,), off + i, i32)],
                        jnp.full((16,), s, f32), mask=lane0)
                eidx = (row0 + b) * 128 + off + lanes
                msk = eidx < E
                a16 = jnp.where(msk, ab[b, pl.ds(off, 16)], 0.0)
                ab[b, pl.ds(off, 16)] = a16
                return jnp.maximum(mx2, jnp.where(msk, a16, -1e30))

            mx = lax.fori_loop(0, 32, grp, mx)
            pltpu.sync_copy(ab, attn_h.at[pl.ds(row0, 4)])
            return mx

        mx = lax.fori_loop(0, 20, chunk, jnp.full((16,), -1e30, f32))
        mbuf[0, pl.ds(0, 16)] = mx
        pltpu.sync_copy(mbuf, mp_h.at[pl.ds(wid, 1)])

    return k(qwb_p, src2, tf)


def _sc_scatter1(attn2, src2, asg2, mpart):
    @functools.partial(
        pl.kernel,
        out_type=(
            jax.ShapeDtypeStruct((2, S1R, 16), f32),
            jax.ShapeDtypeStruct((EPR, 128), f32),
        ),
        mesh=_mesh(),
        compiler_params=pltpu.CompilerParams(use_tc_tiling_on_sc=False, needs_layout_passes=False),
        scratch_types=[
            pltpu.VMEM_SHARED((S1R, 16), f32),
            pltpu.VMEM((8, 128), f32),
            pltpu.VMEM((8, 128), i32),
            pltpu.VMEM((8, 128), i32),
            pltpu.VMEM((8, 128), f32),
            pltpu.VMEM((8, 128), i32),
            pltpu.VMEM((1024, 16), f32),
            pltpu.VMEM((NW, 128), f32),
            pltpu.SemaphoreType.DMA,
        ],
    )
    def k(attn_h, src_h, asg_h, mp_h, s1_h, ex_h,
          s1s, abuf, sbuf, cbuf, exbuf, idxb, rowb, mb, sem):
        cc = lax.axis_index("c")
        ss = lax.axis_index("s")
        wid = cc * 16 + ss
        zero16 = jnp.zeros((16,), f32)
        tmpl = jnp.where(lax.iota(i32, 16) == 1, 1.0, 0.0).astype(f32)

        def zb_init(i, t):
            rowb[i, :] = zero16
            return t

        lax.fori_loop(0, 1024, zb_init, 0)
        for p in range(4):
            pltpu.sync_copy(rowb, s1s.at[pl.ds(ss * 5008 + p * 1024, 1024)])
        pltpu.sync_copy(rowb.at[pl.ds(0, 912)],
                        s1s.at[pl.ds(ss * 5008 + 4096, 912)])

        def rb_init(i, t):
            rowb[i, :] = tmpl
            return t

        lax.fori_loop(0, 1024, rb_init, 0)
        pltpu.sync_copy(mp_h, mb)
        plsc.subcore_barrier()

        def mred(r, mv):
            return jnp.maximum(mv, mb[r, pl.ds(0, 16)])

        mvec16 = lax.fori_loop(0, NW, mred, jnp.full((16,), -1e30, f32))
        M = jnp.max(mvec16)
        lanes = lax.iota(i32, 16)
        zlanes = jnp.zeros((16,), i32)

        def chunk(j, t):
            row0 = wid * 80 + j * 8
            pltpu.sync_copy(attn_h.at[pl.ds(row0, 8)], abuf)
            pltpu.sync_copy(src_h.at[pl.ds(row0, 8)], sbuf)
            pltpu.sync_copy(asg_h.at[pl.ds(row0, 8)], cbuf)
            for g in range(64):
                b, off = g // 8, (g % 8) * 16
                eidx = (row0 + b) * 128 + off + lanes
                msk = eidx < E
                ex = jnp.where(msk, jnp.exp(abuf[b, pl.ds(off, 16)] - M), 0.0)
                exbuf[b, pl.ds(off, 16)] = ex
                idx = sbuf[b, pl.ds(off, 16)] * 8 + cbuf[b, pl.ds(off, 16)]
                idxb[b, pl.ds(off, 16)] = jnp.where(msk, idx, N * NCL)
                plsc.store_scatter(rowb, [g * 16 + lanes, zlanes], ex)
            pltpu.sync_copy(exbuf, ex_h.at[pl.ds(row0, 8)])
            descs = [
                pltpu.async_copy(rowb.at[pl.ds(b * 128, 128)],
                                 s1s.at[idxb.at[b]], sem, add=True)
                for b in range(8)
            ]
            for dsc in descs:
                dsc.wait()
            return t

        lax.fori_loop(0, 10, chunk, 0)
        plsc.subcore_barrier()
        pltpu.sync_copy(s1s.at[pl.ds(ss * 5008, 5008)],
                        s1_h.at[cc, pl.ds(ss * 5008, 5008)])

    return k(attn2, src2, asg2, mpart)


def _scale_body(es_ref, cn_ref, scaleT_ref, bscal_ref):
    es = es_ref[0] + es_ref[1]
    cn = cn_ref[0] + cn_ref[1]
    g = (jnp.sum(cn, axis=1, keepdims=True) > 0).astype(f32)
    nne = jnp.sum(g)
    mc = jnp.maximum(cn, 1.0)
    scaleT_ref[...] = g / (jnp.where(es > 0, es, 1.0) * mc * nne)
    bscal_ref[...] = jnp.sum(
        g * (cn > 0).astype(f32) / (mc * nne), axis=0, keepdims=True)


def _scale(es2, cn2):
    return pl.pallas_call(
        _scale_body,
        out_shape=[
            jax.ShapeDtypeStruct((NCL, N), f32),
            jax.ShapeDtypeStruct((1, N), f32),
        ],
    )(es2, cn2)


def _sc_wgather(scalef, ex2, src2, asg2):
    @functools.partial(
        pl.kernel,
        out_type=jax.ShapeDtypeStruct((EPR, 128), f32),
        mesh=_mesh(),
        compiler_params=pltpu.CompilerParams(use_tc_tiling_on_sc=False, needs_layout_passes=False),
        scratch_types=[
            pltpu.VMEM((S1R,), f32),
            pltpu.VMEM((16, 128), f32),
            pltpu.VMEM((16, 128), i32),
            pltpu.VMEM((16, 128), i32),
            pltpu.VMEM((16, 128), f32),
        ],
    )
    def k(scale_h, ex_h, src_h, asg_h, w_h, scv, exb, sb, cb, wb):
        wid = lax.axis_index("c") * 16 + lax.axis_index("s")
        pltpu.sync_copy(scale_h, scv)

        def chunk(j, t):
            row0 = wid * 80 + j * 16
            pltpu.sync_copy(ex_h.at[pl.ds(row0, 16)], exb)
            pltpu.sync_copy(src_h.at[pl.ds(row0, 16)], sb)
            pltpu.sync_copy(asg_h.at[pl.ds(row0, 16)], cb)
            lanes = lax.iota(i32, 16)
            for g in range(128):
                b, off = g // 8, (g % 8) * 16
                eidx = (row0 + b) * 128 + off + lanes
                idx = sb[b, pl.ds(off, 16)] * 8 + cb[b, pl.ds(off, 16)]
                idx = jnp.where(eidx < E, idx, N * NCL)
                sc16 = plsc.load_gather(scv, [idx])
                wb[b, pl.ds(off, 16)] = exb[b, pl.ds(off, 16)] * sc16
            pltpu.sync_copy(wb, w_h.at[pl.ds(row0, 16)])
            return t

        lax.fori_loop(0, 5, chunk, 0)

    return k(scalef, ex2, src2, asg2)


def _sc_scatter2(w2, src2, tf):
    @functools.partial(
        pl.kernel,
        out_type=jax.ShapeDtypeStruct((2, TR, D), f32),
        mesh=_mesh(),
        compiler_params=pltpu.CompilerParams(use_tc_tiling_on_sc=False, needs_layout_passes=False),
        scratch_types=[
            pltpu.VMEM_SHARED((TR, D), f32),
            pltpu.VMEM((512, D), f32),
            pltpu.VMEM((4, 128), f32),
            pltpu.VMEM((4, 128), i32),
            pltpu.SemaphoreType.DMA,
        ],
    )
    def k(w_h, src_h, tf_h, tot_h, tots, tfb, wb, idxb, sem):
        cc = lax.axis_index("c")
        ss = lax.axis_index("s")
        wid = cc * 16 + ss
        zero16 = jnp.zeros((16,), f32)

        def z_init(i, t):
            for j in range(4):
                tfb[i, pl.ds(j * 16, 16)] = zero16
            return t

        lax.fori_loop(0, 512, z_init, 0)
        pltpu.sync_copy(tfb, tots.at[pl.ds(ss * 626, 512)])
        pltpu.sync_copy(tfb.at[pl.ds(0, 114)],
                        tots.at[pl.ds(ss * 626 + 512, 114)])
        plsc.subcore_barrier()

        def chunk(j, t):
            row0 = wid * 80 + j * 4
            pltpu.sync_copy(tf_h.at[pl.ds(row0 * 128, 512)], tfb)
            pltpu.sync_copy(w_h.at[pl.ds(row0, 4)], wb)
            pltpu.sync_copy(src_h.at[pl.ds(row0, 4)], idxb)

            def pereg(g2, t2):
                w16 = wb[g2 // 8, pl.ds((g2 % 8) * 16, 16)]
                for i in range(16):
                    e = g2 * 16 + i
                    w = w16[i]
                    for j4 in range(4):
                        tfb[e, pl.ds(j4 * 16, 16)] = (
                            tfb[e, pl.ds(j4 * 16, 16)] * w)
                return t2

            lax.fori_loop(0, 32, pereg, 0)
            descs = [
                pltpu.async_copy(tfb.at[pl.ds(b * 128, 128)],
                                 tots.at[idxb.at[b]], sem, add=True)
                for b in range(4)
            ]
            for dsc in descs:
                dsc.wait()
            return t

        lax.fori_loop(0, 20, chunk, 0)
        plsc.subcore_barrier()
        pltpu.sync_copy(tots.at[pl.ds(ss * 626, 626)],
                        tot_h.at[cc, pl.ds(ss * 626, 626)])

    return k(w2, src2, tf)


def _final_body(tot_ref, x_ref, bs_ref, Wv_ref, bv_ref, Wo_ref, bo_ref,
                Wu_ref, bu_ref, y_ref):
    A = tot_ref[0, :N, :] + tot_ref[1, :N, :]
    combined = (jnp.dot(A, Wv_ref[...], precision="highest")
                + bs_ref[...] * bv_ref[...])
    fused = jnp.maximum(
        jnp.dot(combined, Wo_ref[...], precision="highest") + bo_ref[...], 0.0)
    out = jnp.dot(fused, Wu_ref[...], precision="highest") + bu_ref[...]
    y_ref[...] = x_ref[...] + out


def _final(totals, x, bscalT, Wv, bv, Wout, bout, Wup, bup):
    return pl.pallas_call(
        _final_body,
        out_shape=jax.ShapeDtypeStruct((N, 128), f32),
    )(totals, x, bscalT, Wv, bv, Wout, bout, Wup, bup)


def kernel(x, edge_index, edge_attr, Wdown, bdown, Wtime, btime, Wq, bq,
           Wk, bk, Wv, bv, cluster_emb, Wout, bout, Wup, bup):
    src = edge_index[0].astype(i32)
    src2 = jnp.concatenate(
        [src.reshape(E // 128, 128), jnp.full((EPR - E // 128, 128), N, i32)],
        axis=0)

    qwb = _node_dense(x, Wdown, bdown.reshape(1, D), Wq, bq.reshape(1, D),
                      Wk.T, bk.reshape(D, 1))
    qwb_p = jnp.pad(qwb, ((0, 8), (0, 0)))

    tf, asg3 = _edge_dense(edge_attr, Wtime, btime.reshape(1, D), cluster_emb)
    asg2 = jnp.concatenate(
        [asg3.reshape(GB * (EB // 128), 128),
         jnp.zeros((EPR - GB * (EB // 128), 128), i32)], axis=0)
    attn2, mpart = _sc_gatherdot(qwb_p, src2, tf)

    s1, ex2 = _sc_scatter1(attn2, src2, asg2, mpart)

    es2 = s1[:, :N * 8, 0].reshape(2, N, NCL).transpose(0, 2, 1)
    cn2 = s1[:, :N * 8, 1].reshape(2, N, NCL).transpose(0, 2, 1)
    scaleT, bscal = _scale(es2, cn2)
    scalef = jnp.pad(scaleT.T.reshape(N * NCL), (0, S1R - N * NCL))

    w2 = _sc_wgather(scalef, ex2, src2, asg2)
    totals = _sc_scatter2(w2, src2, tf)

    return _final(totals, x, bscal.T, Wv, bv.reshape(1, D),
                  Wout, bout.reshape(1, D), Wup, bup.reshape(1, 128))
